# baseline (speedup 1.0000x reference)
import functools

import jax
import jax.numpy as jnp
from jax import lax
from jax.experimental import pallas as pl
from jax.experimental.pallas import tpu as pltpu
from jax.experimental.pallas import tpu_sc as plsc

_VOCAB = 100000
_EMBED = 64
_BATCH = 4096

_NC = 2
_NS = 16
_NW = _NC * _NS
_B_PER_W = _BATCH // _NW

_VB = 512
_BB = 2048
_NVA = _VOCAB // _VB
_NB = _BATCH // _BB
_NBUF = 6
_TAIL = _VOCAB - _NVA * _VB
_TAIL_BLK = (_NVA * _VB) // _TAIL


@functools.lru_cache(maxsize=1)
def _make_gather():
    mesh = plsc.VectorSubcoreMesh(core_axis_name="c", subcore_axis_name="s")

    @functools.partial(
        pl.kernel,
        mesh=mesh,
        out_type=jax.ShapeDtypeStruct((_BATCH, _EMBED), jnp.float32),
        scratch_types=[
            pltpu.VMEM((_B_PER_W,), jnp.int32),
            pltpu.VMEM((_B_PER_W, _EMBED), jnp.float32),
            pltpu.SemaphoreType.DMA,
        ],
        compiler_params=pltpu.CompilerParams(use_tc_tiling_on_sc=False),
    )
    def gather(table_hbm, idx_hbm, out_hbm, idx_v, rows_v, sem):
        wid = lax.axis_index("s") * _NC + lax.axis_index("c")
        base = wid * _B_PER_W
        pltpu.sync_copy(idx_hbm.at[pl.ds(base, _B_PER_W)], idx_v)
        pltpu.async_copy(table_hbm.at[idx_v], rows_v, sem).wait()
        pltpu.sync_copy(rows_v, out_hbm.at[pl.ds(base, _B_PER_W)])

    return gather


def _mm_body(e_ref, bias_ref, w_ref, o_hbm, acc, sems):
    bi = pl.program_id(0)
    vi = pl.program_id(1)
    step = bi * _NVA + vi
    slot = lax.rem(step, _NBUF)

    @pl.when(step >= _NBUF)
    def _wait_slot():
        pltpu.make_async_copy(
            acc.at[slot], o_hbm.at[pl.ds(0, _BB), pl.ds(0, _VB)], sems.at[slot]
        ).wait()

    w = w_ref[...]
    acc[slot] = (
        lax.dot_general(
            e_ref[...], w, (((1,), (1,)), ((), ())),
            preferred_element_type=jnp.float32,
        )
        + bias_ref[0]
    )
    pltpu.make_async_copy(
        acc.at[slot],
        o_hbm.at[pl.ds(bi * _BB, _BB), pl.ds(vi * _VB, _VB)],
        sems.at[slot],
    ).start()

    @pl.when(step == _NB * _NVA - 1)
    def _drain():
        for k in range(_NBUF):
            sl = (_NB * _NVA - _NBUF + k) % _NBUF
            pltpu.make_async_copy(
                acc.at[sl], o_hbm.at[pl.ds(0, _BB), pl.ds(0, _VB)], sems.at[sl]
            ).wait()


def _tail_body(e_ref, w_ref, b_ref, prev_ref, o_ref):
    del prev_ref
    o_ref[...] = (
        lax.dot_general(
            e_ref[...], w_ref[...], (((1,), (1,)), ((), ())),
            preferred_element_type=jnp.float32,
        )
        + b_ref[...]
    )


def kernel(inputs, emb_table, W, b):
    embeds = _make_gather()(emb_table, inputs)
    bias3d = b[: _NVA * _VB].reshape(_NVA, 1, _VB)
    main = pl.pallas_call(
        _mm_body,
        grid=(_NB, _NVA),
        in_specs=[
            pl.BlockSpec((_BB, _EMBED), lambda bi, vi: (bi, 0)),
            pl.BlockSpec((1, 1, _VB), lambda bi, vi: (vi, 0, 0)),
            pl.BlockSpec((_VB, _EMBED), lambda bi, vi: (vi, 0)),
        ],
        out_specs=pl.BlockSpec(memory_space=pl.ANY),
        out_shape=jax.ShapeDtypeStruct((_BATCH, _VOCAB), jnp.float32),
        scratch_shapes=[
            pltpu.VMEM((_NBUF, _BB, _VB), jnp.float32),
            pltpu.SemaphoreType.DMA((_NBUF,)),
        ],
    )(embeds, bias3d, W)
    w_tail = jnp.pad(W[_NVA * _VB :], ((0, _VB - _TAIL), (0, 0)))
    b_tail = jnp.pad(b[_NVA * _VB :], (0, _VB - _TAIL)).reshape(1, _VB)
    logits = pl.pallas_call(
        _tail_body,
        grid=(_NB,),
        in_specs=[
            pl.BlockSpec((_BB, _EMBED), lambda bi: (bi, 0)),
            pl.BlockSpec((_VB, _EMBED), lambda bi: (0, 0)),
            pl.BlockSpec((1, _VB), lambda bi: (0, 0)),
            pl.BlockSpec(memory_space=pl.ANY),
        ],
        out_specs=pl.BlockSpec((_BB, _VB), lambda bi: (bi, _NVA)),
        out_shape=jax.ShapeDtypeStruct((_BATCH, _VOCAB), jnp.float32),
        input_output_aliases={3: 0},
    )(embeds, w_tail, b_tail, main)
    return logits


# speedup vs baseline: 3.0222x; 3.0222x over previous
import functools

import jax
import jax.numpy as jnp
from jax import lax
from jax.experimental import pallas as pl
from jax.experimental.pallas import tpu as pltpu
from jax.experimental.pallas import tpu_sc as plsc

_VOCAB = 100000
_EMBED = 64
_BATCH = 4096

_NC = 2
_NS = 16
_NW = _NC * _NS
_B_PER_W = _BATCH // _NW

_VB = 512
_NV = (_VOCAB + _VB - 1) // _VB


@functools.lru_cache(maxsize=1)
def _make_gather():
    mesh = plsc.VectorSubcoreMesh(core_axis_name="c", subcore_axis_name="s")

    @functools.partial(
        pl.kernel,
        mesh=mesh,
        out_type=jax.ShapeDtypeStruct((_BATCH, _EMBED), jnp.float32),
        scratch_types=[
            pltpu.VMEM((_B_PER_W,), jnp.int32),
            pltpu.VMEM((_B_PER_W, _EMBED), jnp.float32),
            pltpu.SemaphoreType.DMA,
        ],
        compiler_params=pltpu.CompilerParams(use_tc_tiling_on_sc=False),
    )
    def gather(table_hbm, idx_hbm, out_hbm, idx_v, rows_v, sem):
        wid = lax.axis_index("s") * _NC + lax.axis_index("c")
        base = wid * _B_PER_W
        pltpu.sync_copy(idx_hbm.at[pl.ds(base, _B_PER_W)], idx_v)
        pltpu.async_copy(table_hbm.at[idx_v], rows_v, sem).wait()
        pltpu.sync_copy(rows_v, out_hbm.at[pl.ds(base, _B_PER_W)])

    return gather


def _mm_body(w_ref, e_ref, b_ref, o_ref):
    o_ref[...] = (
        lax.dot_general(
            w_ref[...], e_ref[...], (((1,), (1,)), ((), ())),
            preferred_element_type=jnp.float32,
        )
        + b_ref[...]
    )


def kernel(inputs, emb_table, W, b):
    embeds = _make_gather()(emb_table, inputs)
    logits_t = pl.pallas_call(
        _mm_body,
        grid=(_NV,),
        in_specs=[
            pl.BlockSpec((_VB, _EMBED), lambda i: (i, 0)),
            pl.BlockSpec((_BATCH, _EMBED), lambda i: (0, 0)),
            pl.BlockSpec((_VB, 1), lambda i: (i, 0)),
        ],
        out_specs=pl.BlockSpec((_VB, _BATCH), lambda i: (i, 0)),
        out_shape=jax.ShapeDtypeStruct((_VOCAB, _BATCH), jnp.float32),
    )(W, embeds, b.reshape(_VOCAB, 1))
    return logits_t.T


# speedup vs baseline: 3.0762x; 1.0179x over previous
import functools

import jax
import jax.numpy as jnp
from jax import lax
from jax.experimental import pallas as pl
from jax.experimental.pallas import tpu as pltpu
from jax.experimental.pallas import tpu_sc as plsc

_VOCAB = 100000
_EMBED = 64
_BATCH = 4096

_NC = 2
_NS = 16
_NW = _NC * _NS
_B_PER_W = _BATCH // _NW

_VB = 512
_NV = (_VOCAB + _VB - 1) // _VB
_NBUF = 4
_TAIL = _VOCAB - (_NV - 1) * _VB


@functools.lru_cache(maxsize=1)
def _make_gather():
    mesh = plsc.VectorSubcoreMesh(core_axis_name="c", subcore_axis_name="s")

    @functools.partial(
        pl.kernel,
        mesh=mesh,
        out_type=jax.ShapeDtypeStruct((_BATCH, _EMBED), jnp.float32),
        scratch_types=[
            pltpu.VMEM((_B_PER_W,), jnp.int32),
            pltpu.VMEM((_B_PER_W, _EMBED), jnp.float32),
            pltpu.SemaphoreType.DMA,
        ],
        compiler_params=pltpu.CompilerParams(use_tc_tiling_on_sc=False),
    )
    def gather(table_hbm, idx_hbm, out_hbm, idx_v, rows_v, sem):
        wid = lax.axis_index("s") * _NC + lax.axis_index("c")
        base = wid * _B_PER_W
        pltpu.sync_copy(idx_hbm.at[pl.ds(base, _B_PER_W)], idx_v)
        pltpu.async_copy(table_hbm.at[idx_v], rows_v, sem).wait()
        pltpu.sync_copy(rows_v, out_hbm.at[pl.ds(base, _B_PER_W)])

    return gather


def _mm_body(w_ref, e_ref, b_ref, o_hbm, acc, sems):
    i = pl.program_id(0)
    slot = lax.rem(i, _NBUF)

    @pl.when(i >= _NBUF)
    def _wait_slot():
        pltpu.make_async_copy(
            acc.at[slot], o_hbm.at[pl.ds(0, _VB)], sems.at[slot]
        ).wait()

    acc[slot] = (
        lax.dot_general(
            w_ref[...], e_ref[...], (((1,), (1,)), ((), ())),
            preferred_element_type=jnp.float32,
        )
        + b_ref[...]
    )

    @pl.when(i < _NV - 1)
    def _copy_full():
        pltpu.make_async_copy(
            acc.at[slot], o_hbm.at[pl.ds(i * _VB, _VB)], sems.at[slot]
        ).start()

    @pl.when(i == _NV - 1)
    def _copy_tail():
        pltpu.make_async_copy(
            acc.at[slot, pl.ds(0, _TAIL)],
            o_hbm.at[pl.ds((_NV - 1) * _VB, _TAIL)],
            sems.at[slot],
        ).start()

    @pl.when(i == _NV - 1)
    def _drain():
        for k in range(_NBUF):
            s = _NV - _NBUF + k
            sl = s % _NBUF
            if s == _NV - 1:
                pltpu.make_async_copy(
                    acc.at[sl, pl.ds(0, _TAIL)],
                    o_hbm.at[pl.ds(0, _TAIL)],
                    sems.at[sl],
                ).wait()
            else:
                pltpu.make_async_copy(
                    acc.at[sl], o_hbm.at[pl.ds(0, _VB)], sems.at[sl]
                ).wait()


def kernel(inputs, emb_table, W, b):
    embeds = _make_gather()(emb_table, inputs)
    logits_t = pl.pallas_call(
        _mm_body,
        grid=(_NV,),
        in_specs=[
            pl.BlockSpec((_VB, _EMBED), lambda i: (i, 0)),
            pl.BlockSpec((_BATCH, _EMBED), lambda i: (0, 0)),
            pl.BlockSpec((_VB, 1), lambda i: (i, 0)),
        ],
        out_specs=pl.BlockSpec(memory_space=pl.ANY),
        out_shape=jax.ShapeDtypeStruct((_VOCAB, _BATCH), jnp.float32),
        scratch_shapes=[
            pltpu.VMEM((_NBUF, _VB, _BATCH), jnp.float32),
            pltpu.SemaphoreType.DMA((_NBUF,)),
        ],
    )(W, embeds, b.reshape(_VOCAB, 1))
    return logits_t.T
